# baseline (speedup 1.0000x reference)
import functools

import jax
import jax.numpy as jnp
from jax import lax
from jax.experimental import pallas as pl
from jax.experimental.pallas import tpu as pltpu
from jax.experimental.pallas import tpu_sc as plsc

_N = 10000
_F = 128
_K = 6
_G = 16
_NP = 10240
_INF = float("inf")
_BIG = 2 ** 30


_BR_PRE = 2000


def _pre_body(x_ref, w1_ref, b1_ref, a1_ref, w2_ref, b2_ref, a2_ref,
              h_ref, st_ref):
    i = pl.program_id(0)
    xb = x_ref[...]
    h1 = jnp.dot(xb, w1_ref[...], preferred_element_type=jnp.float32) + b1_ref[...]
    h1 = jnp.where(h1 >= 0, h1, a1_ref[...] * h1)
    h2 = jnp.dot(h1, w2_ref[...], preferred_element_type=jnp.float32) + b2_ref[...]
    h2 = jnp.where(h2 >= 0, h2, a2_ref[...] * h2)
    h_ref[...] = h2
    s = jnp.sum(h2, axis=0, keepdims=True)
    ss = jnp.sum(h2 * h2, axis=0, keepdims=True)
    row = lax.broadcasted_iota(jnp.int32, (8, _F), 0)
    upd = jnp.where(row == 0, s, 0.0) + jnp.where(row == 1, ss, 0.0)

    @pl.when(i == 0)
    def _():
        st_ref[...] = upd

    @pl.when(i > 0)
    def _():
        st_ref[...] = st_ref[...] + upd


def _pre_call(x, w1, b1, a1, w2, b2, a2):
    nb = _N // _BR_PRE
    cspec = lambda shape: pl.BlockSpec(shape, lambda i: (0, 0))
    return pl.pallas_call(
        _pre_body,
        grid=(nb,),
        in_specs=[
            pl.BlockSpec((_BR_PRE, _F), lambda i: (i, 0)),
            cspec((_F, _F)), cspec((1, _F)), cspec((1, _F)),
            cspec((_F, _F)), cspec((1, _F)), cspec((1, _F)),
        ],
        out_specs=[
            pl.BlockSpec((_BR_PRE, _F), lambda i: (i, 0)),
            pl.BlockSpec((8, _F), lambda i: (0, 0)),
        ],
        out_shape=[
            jax.ShapeDtypeStruct((_N, _F), jnp.float32),
            jax.ShapeDtypeStruct((8, _F), jnp.float32),
        ],
    )(x, w1, b1, a1, w2, b2, a2)


_BR_BN = 2000


def _bn_body(hp_ref, st_ref, gm_ref, bt_ref, bidx_ref, h0_ref, pool_ref):
    i = pl.program_id(0)
    st = st_ref[...]
    mu = st[0:1, :] * (1.0 / _N)
    ex2 = st[1:2, :] * (1.0 / _N)
    var = ex2 - mu * mu
    scale = lax.rsqrt(var + 1e-5) * gm_ref[...]
    h0 = (hp_ref[...] - mu) * scale + bt_ref[...]
    h0_ref[...] = h0
    b = bidx_ref[0]
    oh = (b == lax.broadcasted_iota(jnp.int32, (_G, _BR_BN), 0)).astype(jnp.float32)
    pool = lax.dot_general(oh, h0, (((1,), (0,)), ((), ())),
                           preferred_element_type=jnp.float32)

    @pl.when(i == 0)
    def _():
        pool_ref[...] = pool

    @pl.when(i > 0)
    def _():
        pool_ref[...] = pool_ref[...] + pool


def _bn_call(h_pre, stats, gamma, beta, batch3):
    nb = _N // _BR_BN
    return pl.pallas_call(
        _bn_body,
        grid=(nb,),
        in_specs=[
            pl.BlockSpec((_BR_BN, _F), lambda i: (i, 0)),
            pl.BlockSpec((8, _F), lambda i: (0, 0)),
            pl.BlockSpec((1, _F), lambda i: (0, 0)),
            pl.BlockSpec((1, _F), lambda i: (0, 0)),
            pl.BlockSpec((1, 1, _BR_BN), lambda i: (i, 0, 0)),
        ],
        out_specs=[
            pl.BlockSpec((_BR_BN, _F), lambda i: (i, 0)),
            pl.BlockSpec((_G, _F), lambda i: (0, 0)),
        ],
        out_shape=[
            jax.ShapeDtypeStruct((_N, _F), jnp.float32),
            jax.ShapeDtypeStruct((_G, _F), jnp.float32),
        ],
    )(h_pre, stats, gamma, beta, batch3)


_BRK = 256
_BCK = 1024
_NCJ = _NP // _BCK


_FBIG = 1e9


def _knn_body(rng_ref, xr_ref, xt_ref, sqr_ref, sqc_ref, brow_ref, bcol_ref,
              nbr_ref, rv_ref, ri_ref):
    i = pl.program_id(0)
    j = pl.program_id(1)

    @pl.when(j == 0)
    def _():
        rv_ref[...] = jnp.full((_BRK, 8), _INF, jnp.float32)
        ri_ref[...] = jnp.full((_BRK, 8), _FBIG, jnp.float32)

    nbi = _NP // _BRK
    rel = ((rng_ref[2 * nbi + j] <= rng_ref[nbi + i])
           & (rng_ref[2 * nbi + _NCJ + j] >= rng_ref[i]))

    @pl.when(rel)
    def _():
        xr = xr_ref[...]
        xt = xt_ref[...]
        sqr = sqr_ref[...]
        sqc = sqc_ref[...]
        d = lax.dot_general(xr, xt, (((1,), (0,)), ((), ())),
                            preferred_element_type=jnp.float32)
        d2 = (sqr + sqc) + d
        coli = lax.broadcasted_iota(jnp.int32, (_BRK, _BCK), 1)
        rowi = lax.broadcasted_iota(jnp.int32, (_BRK, _BCK), 0)
        colg = (coli + j * _BCK).astype(jnp.float32)
        mask = (brow_ref[...] != bcol_ref[...]) | ((coli - rowi) == (i * _BRK - j * _BCK))
        d2 = jnp.where(mask, _INF, d2)

        cv, ci = [], []
        for _ in range(_K):
            m = jnp.min(d2, axis=1, keepdims=True)
            sel = jnp.where(d2 == m, colg, _FBIG)
            am = jnp.min(sel, axis=1, keepdims=True)
            cv.append(m)
            ci.append(am)
            d2 = jnp.where(sel == am, _INF, d2)

        allv = jnp.concatenate(cv + [rv_ref[...]], axis=1)
        alli = jnp.concatenate(ci + [ri_ref[...]], axis=1)
        nv, ni = [], []
        for _ in range(_K):
            m = jnp.min(allv, axis=1, keepdims=True)
            sel = jnp.where(allv == m, alli, _FBIG)
            am = jnp.min(sel, axis=1, keepdims=True)
            nv.append(m)
            ni.append(am)
            allv = jnp.where(sel == am, _INF, allv)

        rv_ref[:, 0:_K] = jnp.concatenate(nv, axis=1)
        ri_ref[:, 0:_K] = jnp.concatenate(ni, axis=1)

    @pl.when(j == _NCJ - 1)
    def _():
        ri = ri_ref[...]
        idx8 = jnp.concatenate(
            [ri[:, 0:_K], ri[:, _K - 1:_K], ri[:, _K - 1:_K]], axis=1)
        nbr_ref[...] = jnp.minimum(idx8, _N - 1).astype(jnp.int32)


def _knn_call(xp, xt, sqr, sqc, brow, bcol, rng):
    nbi = _NP // _BRK
    return pl.pallas_call(
        _knn_body,
        grid_spec=pltpu.PrefetchScalarGridSpec(
            num_scalar_prefetch=1,
            grid=(nbi, _NCJ),
            in_specs=[
                pl.BlockSpec((_BRK, _F), lambda i, j, rng: (i, 0)),
                pl.BlockSpec((_F, _BCK), lambda i, j, rng: (0, j)),
                pl.BlockSpec((_BRK, 1), lambda i, j, rng: (i, 0)),
                pl.BlockSpec((1, _BCK), lambda i, j, rng: (0, j)),
                pl.BlockSpec((_BRK, 1), lambda i, j, rng: (i, 0)),
                pl.BlockSpec((1, _BCK), lambda i, j, rng: (0, j)),
            ],
            out_specs=pl.BlockSpec((_BRK, 8), lambda i, j, rng: (i, 0)),
            scratch_shapes=[
                pltpu.VMEM((_BRK, 8), jnp.float32),
                pltpu.VMEM((_BRK, 8), jnp.float32),
            ],
        ),
        out_shape=jax.ShapeDtypeStruct((_NP, 8), jnp.int32),
    )(rng, xp, xt, sqr, sqc, brow, bcol)


_NW = 32
_BW = _NP // _NW
_GN = 16
_GROWS = _GN * _K
_NCH = _BW // _GN


def _gather_sum_sc(h, nbr3):
    mesh = plsc.VectorSubcoreMesh(core_axis_name="c", subcore_axis_name="s")

    @functools.partial(
        pl.kernel,
        out_type=jax.ShapeDtypeStruct((_NP, _F), jnp.float32),
        mesh=mesh,
        scratch_types=[
            pltpu.VMEM((_NCH, _GROWS), jnp.int32),
            pltpu.VMEM((4, _GROWS, _F), jnp.float32),
            pltpu.VMEM((_BW, _F), jnp.float32),
            pltpu.SemaphoreType.DMA,
            pltpu.SemaphoreType.DMA,
            pltpu.SemaphoreType.DMA,
            pltpu.SemaphoreType.DMA,
        ],
    )
    def gk(h_hbm, nbr_hbm, out_hbm, idx_v, rows_v, acc_v, sem0, sem1, sem2, sem3):
        wid = lax.axis_index("s") * 2 + lax.axis_index("c")
        base = wid * _BW
        pltpu.sync_copy(nbr_hbm.at[wid], idx_v)
        sems = [sem0, sem1, sem2, sem3]
        ring = 4
        handles = [None] * ring
        for p in range(ring):
            handles[p] = pltpu.async_copy(
                h_hbm.at[idx_v.at[p]], rows_v.at[p], sems[p])
        for c in range(_NCH):
            b = c % ring
            handles[b].wait()

            def body(nn, _, _b=b, _c=c):
                r0 = nn * _K
                for f8 in range(_F // 16):
                    sl = pl.ds(f8 * 16, 16)
                    v = rows_v[_b, r0, sl]
                    for k in range(1, _K):
                        v = v + rows_v[_b, r0 + k, sl]
                    acc_v[_c * _GN + nn, sl] = v
                return 0

            lax.fori_loop(0, _GN, body, 0)
            if c + ring < _NCH:
                handles[b] = pltpu.async_copy(
                    h_hbm.at[idx_v.at[c + ring]], rows_v.at[b], sems[b])
        pltpu.sync_copy(acc_v, out_hbm.at[pl.ds(base, _BW)])

    return gk(h, nbr3)


_BR_CV = 2000


def _conv_body(g_ref, h_ref, w_ref, b_ref, a_ref, bidx_ref, hn_ref, pool_ref):
    i = pl.program_id(0)
    agg = jnp.dot(g_ref[...], w_ref[...], preferred_element_type=jnp.float32)
    hn = agg + (_K * 1.0) * b_ref[...] + h_ref[...]
    a = a_ref[...]
    hn = jnp.where(hn >= 0, hn, a * hn)
    hn_ref[...] = hn
    b = bidx_ref[0]
    oh = (b == lax.broadcasted_iota(jnp.int32, (_G, _BR_CV), 0)).astype(jnp.float32)
    pool = lax.dot_general(oh, hn, (((1,), (0,)), ((), ())),
                           preferred_element_type=jnp.float32)

    @pl.when(i == 0)
    def _():
        pool_ref[...] = pool

    @pl.when(i > 0)
    def _():
        pool_ref[...] = pool_ref[...] + pool


def _conv_call(g, h, w, bb, aa, batch3):
    nb = _N // _BR_CV
    return pl.pallas_call(
        _conv_body,
        grid=(nb,),
        in_specs=[
            pl.BlockSpec((_BR_CV, _F), lambda i: (i, 0)),
            pl.BlockSpec((_BR_CV, _F), lambda i: (i, 0)),
            pl.BlockSpec((_F, _F), lambda i: (0, 0)),
            pl.BlockSpec((1, _F), lambda i: (0, 0)),
            pl.BlockSpec((1, _F), lambda i: (0, 0)),
            pl.BlockSpec((1, 1, _BR_CV), lambda i: (i, 0, 0)),
        ],
        out_specs=[
            pl.BlockSpec((_BR_CV, _F), lambda i: (i, 0)),
            pl.BlockSpec((_G, _F), lambda i: (0, 0)),
        ],
        out_shape=[
            jax.ShapeDtypeStruct((_N, _F), jnp.float32),
            jax.ShapeDtypeStruct((_G, _F), jnp.float32),
        ],
    )(g, h, w, bb, aa, batch3)


def _ffn_body(p_ref, w1_ref, b1_ref, w2_ref, b2_ref, o_ref):
    z = jnp.dot(p_ref[...], w1_ref[...], preferred_element_type=jnp.float32)
    z = jnp.maximum(z + b1_ref[...], 0.0)
    o_ref[...] = jnp.sum(z * w2_ref[...], axis=1, keepdims=True) + b2_ref[...]


def _ffn_call(pc, w1, b1, w2t, b2):
    d = _F * 5
    return pl.pallas_call(
        _ffn_body,
        in_specs=[
            pl.BlockSpec((_G, d), lambda: (0, 0)),
            pl.BlockSpec((d, d), lambda: (0, 0)),
            pl.BlockSpec((1, d), lambda: (0, 0)),
            pl.BlockSpec((1, d), lambda: (0, 0)),
            pl.BlockSpec((1, 1), lambda: (0, 0)),
        ],
        out_specs=pl.BlockSpec((_G, 1), lambda: (0, 0)),
        out_shape=jax.ShapeDtypeStruct((_G, 1), jnp.float32),
    )(pc, w1, b1, w2t, b2)


def kernel(x, batch, pre_W1, pre_b1, pre_a1, pre_W2, pre_b2, pre_a2,
           bn_gamma, bn_beta, act_a,
           conv0_W, conv0_b, conv1_W, conv1_b, conv2_W, conv2_b,
           conv3_W, conv3_b, ffn_W1, ffn_b1, ffn_W2, ffn_b2):
    batch = batch.astype(jnp.int32)
    r1 = lambda v: v.reshape(1, _F)

    xp = jnp.pad(x, ((0, _NP - _N), (0, 0)))
    xt = (-2.0 * xp).T
    brow = jnp.pad(batch, (0, _NP - _N), constant_values=-2).reshape(_NP, 1)
    bcol = jnp.pad(batch, (0, _NP - _N), constant_values=-1).reshape(1, _NP)
    batch3 = batch.reshape(_N // _BR_CV, 1, _BR_CV)

    sq = jnp.pad(jnp.sum(x * x, axis=1), (0, _NP - _N))

    bpad = jnp.pad(batch, (0, _NP - _N), constant_values=-1)
    blo = jnp.where(bpad >= 0, bpad, _BIG)
    bhi = jnp.where(bpad >= 0, bpad, -_BIG)
    rng = jnp.concatenate([
        jnp.min(blo.reshape(_NP // _BRK, _BRK), axis=1),
        jnp.max(bhi.reshape(_NP // _BRK, _BRK), axis=1),
        jnp.min(blo.reshape(_NCJ, _BCK), axis=1),
        jnp.max(bhi.reshape(_NCJ, _BCK), axis=1),
    ])

    nbr8 = _knn_call(xp, xt, sq.reshape(_NP, 1), sq.reshape(1, _NP),
                     brow, bcol, rng)
    nbr3 = nbr8[:, :_K].reshape(_NW, _NCH, _GROWS)

    h_pre, stats = _pre_call(x, pre_W1, r1(pre_b1), r1(pre_a1),
                             pre_W2, r1(pre_b2), r1(pre_a2))
    h, pool0 = _bn_call(h_pre, stats, r1(bn_gamma), r1(bn_beta), batch3)

    pools = [pool0]
    aa = r1(act_a)
    for w, bb in ((conv0_W, conv0_b), (conv1_W, conv1_b),
                  (conv2_W, conv2_b), (conv3_W, conv3_b)):
        g = _gather_sum_sc(h, nbr3)[:_N]
        h, pool = _conv_call(g, h, w, r1(bb), aa, batch3)
        pools.append(pool)

    pc = jnp.concatenate(pools, axis=1)
    return _ffn_call(pc, ffn_W1, ffn_b1.reshape(1, 5 * _F),
                     ffn_W2.reshape(1, 5 * _F), ffn_b2.reshape(1, 1))


# speedup vs baseline: 1.0983x; 1.0983x over previous
import functools

import jax
import jax.numpy as jnp
from jax import lax
from jax.experimental import pallas as pl
from jax.experimental.pallas import tpu as pltpu
from jax.experimental.pallas import tpu_sc as plsc

_N = 10000
_F = 128
_K = 6
_G = 16
_NP = 10240
_INF = float("inf")
_BIG = 2 ** 30


_BR_PRE = 2000


def _pre_body(x_ref, w1_ref, b1_ref, a1_ref, w2_ref, b2_ref, a2_ref,
              h_ref, st_ref):
    i = pl.program_id(0)
    xb = x_ref[...]
    h1 = jnp.dot(xb, w1_ref[...], preferred_element_type=jnp.float32) + b1_ref[...]
    h1 = jnp.where(h1 >= 0, h1, a1_ref[...] * h1)
    h2 = jnp.dot(h1, w2_ref[...], preferred_element_type=jnp.float32) + b2_ref[...]
    h2 = jnp.where(h2 >= 0, h2, a2_ref[...] * h2)
    h_ref[...] = h2
    s = jnp.sum(h2, axis=0, keepdims=True)
    ss = jnp.sum(h2 * h2, axis=0, keepdims=True)
    row = lax.broadcasted_iota(jnp.int32, (8, _F), 0)
    upd = jnp.where(row == 0, s, 0.0) + jnp.where(row == 1, ss, 0.0)

    @pl.when(i == 0)
    def _():
        st_ref[...] = upd

    @pl.when(i > 0)
    def _():
        st_ref[...] = st_ref[...] + upd


def _pre_call(x, w1, b1, a1, w2, b2, a2):
    nb = _N // _BR_PRE
    cspec = lambda shape: pl.BlockSpec(shape, lambda i: (0, 0))
    return pl.pallas_call(
        _pre_body,
        grid=(nb,),
        in_specs=[
            pl.BlockSpec((_BR_PRE, _F), lambda i: (i, 0)),
            cspec((_F, _F)), cspec((1, _F)), cspec((1, _F)),
            cspec((_F, _F)), cspec((1, _F)), cspec((1, _F)),
        ],
        out_specs=[
            pl.BlockSpec((_BR_PRE, _F), lambda i: (i, 0)),
            pl.BlockSpec((8, _F), lambda i: (0, 0)),
        ],
        out_shape=[
            jax.ShapeDtypeStruct((_N, _F), jnp.float32),
            jax.ShapeDtypeStruct((8, _F), jnp.float32),
        ],
    )(x, w1, b1, a1, w2, b2, a2)


_BR_BN = 2000


def _bn_body(hp_ref, st_ref, gm_ref, bt_ref, bidx_ref, h0_ref, pool_ref):
    i = pl.program_id(0)
    st = st_ref[...]
    mu = st[0:1, :] * (1.0 / _N)
    ex2 = st[1:2, :] * (1.0 / _N)
    var = ex2 - mu * mu
    scale = lax.rsqrt(var + 1e-5) * gm_ref[...]
    h0 = (hp_ref[...] - mu) * scale + bt_ref[...]
    h0_ref[...] = h0
    b = bidx_ref[0]
    oh = (b == lax.broadcasted_iota(jnp.int32, (_G, _BR_BN), 0)).astype(jnp.float32)
    pool = lax.dot_general(oh, h0, (((1,), (0,)), ((), ())),
                           preferred_element_type=jnp.float32)

    @pl.when(i == 0)
    def _():
        pool_ref[...] = pool

    @pl.when(i > 0)
    def _():
        pool_ref[...] = pool_ref[...] + pool


def _bn_call(h_pre, stats, gamma, beta, batch3):
    nb = _N // _BR_BN
    return pl.pallas_call(
        _bn_body,
        grid=(nb,),
        in_specs=[
            pl.BlockSpec((_BR_BN, _F), lambda i: (i, 0)),
            pl.BlockSpec((8, _F), lambda i: (0, 0)),
            pl.BlockSpec((1, _F), lambda i: (0, 0)),
            pl.BlockSpec((1, _F), lambda i: (0, 0)),
            pl.BlockSpec((1, 1, _BR_BN), lambda i: (i, 0, 0)),
        ],
        out_specs=[
            pl.BlockSpec((_BR_BN, _F), lambda i: (i, 0)),
            pl.BlockSpec((_G, _F), lambda i: (0, 0)),
        ],
        out_shape=[
            jax.ShapeDtypeStruct((_N, _F), jnp.float32),
            jax.ShapeDtypeStruct((_G, _F), jnp.float32),
        ],
    )(h_pre, stats, gamma, beta, batch3)


_BRK = 256
_BCK = 2048
_NCJ = _NP // _BCK


_FBIG = 1e9


def _knn_body(rng_ref, xr_ref, xt_ref, sqr_ref, sqc_ref, brow_ref, bcol_ref,
              nbr_ref, rv_ref, ri_ref):
    i = pl.program_id(0)
    j = pl.program_id(1)

    @pl.when(j == 0)
    def _():
        rv_ref[...] = jnp.full((_BRK, 8), _INF, jnp.float32)
        ri_ref[...] = jnp.full((_BRK, 8), _FBIG, jnp.float32)

    nbi = _NP // _BRK
    rel = ((rng_ref[2 * nbi + j] <= rng_ref[nbi + i])
           & (rng_ref[2 * nbi + _NCJ + j] >= rng_ref[i]))

    @pl.when(rel)
    def _():
        xr = xr_ref[...]
        xt = xt_ref[...]
        sqr = sqr_ref[...]
        sqc = sqc_ref[...]
        d = lax.dot_general(xr, xt, (((1,), (0,)), ((), ())),
                            preferred_element_type=jnp.float32)
        d2 = (sqr + sqc) + d
        coli = lax.broadcasted_iota(jnp.int32, (_BRK, _BCK), 1)
        rowi = lax.broadcasted_iota(jnp.int32, (_BRK, _BCK), 0)
        colg = (coli + j * _BCK).astype(jnp.float32)
        mask = (brow_ref[...] != bcol_ref[...]) | ((coli - rowi) == (i * _BRK - j * _BCK))
        d2 = jnp.where(mask, _INF, d2)

        cv, ci = [], []
        for _ in range(_K):
            m = jnp.min(d2, axis=1, keepdims=True)
            sel = jnp.where(d2 == m, colg, _FBIG)
            am = jnp.min(sel, axis=1, keepdims=True)
            cv.append(m)
            ci.append(am)
            d2 = jnp.where(sel == am, _INF, d2)

        allv = jnp.concatenate(cv + [rv_ref[...]], axis=1)
        alli = jnp.concatenate(ci + [ri_ref[...]], axis=1)
        nv, ni = [], []
        for _ in range(_K):
            m = jnp.min(allv, axis=1, keepdims=True)
            sel = jnp.where(allv == m, alli, _FBIG)
            am = jnp.min(sel, axis=1, keepdims=True)
            nv.append(m)
            ni.append(am)
            allv = jnp.where(sel == am, _INF, allv)

        rv_ref[:, 0:_K] = jnp.concatenate(nv, axis=1)
        ri_ref[:, 0:_K] = jnp.concatenate(ni, axis=1)

    @pl.when(j == _NCJ - 1)
    def _():
        ri = ri_ref[...]
        idx8 = jnp.concatenate(
            [ri[:, 0:_K], ri[:, _K - 1:_K], ri[:, _K - 1:_K]], axis=1)
        nbr_ref[...] = jnp.minimum(idx8, _N - 1).astype(jnp.int32)


def _knn_call(xp, xt, sqr, sqc, brow, bcol, rng):
    nbi = _NP // _BRK
    return pl.pallas_call(
        _knn_body,
        grid_spec=pltpu.PrefetchScalarGridSpec(
            num_scalar_prefetch=1,
            grid=(nbi, _NCJ),
            in_specs=[
                pl.BlockSpec((_BRK, _F), lambda i, j, rng: (i, 0)),
                pl.BlockSpec((_F, _BCK), lambda i, j, rng: (0, j)),
                pl.BlockSpec((_BRK, 1), lambda i, j, rng: (i, 0)),
                pl.BlockSpec((1, _BCK), lambda i, j, rng: (0, j)),
                pl.BlockSpec((_BRK, 1), lambda i, j, rng: (i, 0)),
                pl.BlockSpec((1, _BCK), lambda i, j, rng: (0, j)),
            ],
            out_specs=pl.BlockSpec((_BRK, 8), lambda i, j, rng: (i, 0)),
            scratch_shapes=[
                pltpu.VMEM((_BRK, 8), jnp.float32),
                pltpu.VMEM((_BRK, 8), jnp.float32),
            ],
        ),
        out_shape=jax.ShapeDtypeStruct((_NP, 8), jnp.int32),
    )(rng, xp, xt, sqr, sqc, brow, bcol)


_NW = 32
_BW = _NP // _NW
_GN = 20
_GROWS = _GN * _K
_NCH = _BW // _GN


def _gather_sum_sc(h, nbr3):
    mesh = plsc.VectorSubcoreMesh(core_axis_name="c", subcore_axis_name="s")

    @functools.partial(
        pl.kernel,
        out_type=jax.ShapeDtypeStruct((_NP, _F), jnp.float32),
        mesh=mesh,
        scratch_types=[
            pltpu.VMEM((_NCH, _GROWS), jnp.int32),
            pltpu.VMEM((4, _GROWS, _F), jnp.float32),
            pltpu.VMEM((_BW, _F), jnp.float32),
            pltpu.SemaphoreType.DMA,
            pltpu.SemaphoreType.DMA,
            pltpu.SemaphoreType.DMA,
            pltpu.SemaphoreType.DMA,
        ],
    )
    def gk(h_hbm, nbr_hbm, out_hbm, idx_v, rows_v, acc_v, sem0, sem1, sem2, sem3):
        wid = lax.axis_index("s") * 2 + lax.axis_index("c")
        base = wid * _BW
        pltpu.sync_copy(nbr_hbm.at[wid], idx_v)
        sems = [sem0, sem1, sem2, sem3]
        ring = 4
        handles = [None] * ring
        for p in range(ring):
            handles[p] = pltpu.async_copy(
                h_hbm.at[idx_v.at[p]], rows_v.at[p], sems[p])
        for c in range(_NCH):
            b = c % ring
            handles[b].wait()

            def body(nn, _, _b=b, _c=c):
                r0 = nn * _K
                for f8 in range(_F // 16):
                    sl = pl.ds(f8 * 16, 16)
                    v = rows_v[_b, r0, sl]
                    for k in range(1, _K):
                        v = v + rows_v[_b, r0 + k, sl]
                    acc_v[_c * _GN + nn, sl] = v
                return 0

            lax.fori_loop(0, _GN, body, 0)
            if c + ring < _NCH:
                handles[b] = pltpu.async_copy(
                    h_hbm.at[idx_v.at[c + ring]], rows_v.at[b], sems[b])
        pltpu.sync_copy(acc_v, out_hbm.at[pl.ds(base, _BW)])

    return gk(h, nbr3)


_BR_CV = 2000


def _conv_body(g_ref, h_ref, w_ref, b_ref, a_ref, bidx_ref, hn_ref, pool_ref):
    i = pl.program_id(0)
    agg = jnp.dot(g_ref[...], w_ref[...], preferred_element_type=jnp.float32)
    hn = agg + (_K * 1.0) * b_ref[...] + h_ref[...]
    a = a_ref[...]
    hn = jnp.where(hn >= 0, hn, a * hn)
    hn_ref[...] = hn
    b = bidx_ref[0]
    oh = (b == lax.broadcasted_iota(jnp.int32, (_G, _BR_CV), 0)).astype(jnp.float32)
    pool = lax.dot_general(oh, hn, (((1,), (0,)), ((), ())),
                           preferred_element_type=jnp.float32)

    @pl.when(i == 0)
    def _():
        pool_ref[...] = pool

    @pl.when(i > 0)
    def _():
        pool_ref[...] = pool_ref[...] + pool


def _conv_call(g, h, w, bb, aa, batch3):
    nb = _N // _BR_CV
    return pl.pallas_call(
        _conv_body,
        grid=(nb,),
        in_specs=[
            pl.BlockSpec((_BR_CV, _F), lambda i: (i, 0)),
            pl.BlockSpec((_BR_CV, _F), lambda i: (i, 0)),
            pl.BlockSpec((_F, _F), lambda i: (0, 0)),
            pl.BlockSpec((1, _F), lambda i: (0, 0)),
            pl.BlockSpec((1, _F), lambda i: (0, 0)),
            pl.BlockSpec((1, 1, _BR_CV), lambda i: (i, 0, 0)),
        ],
        out_specs=[
            pl.BlockSpec((_BR_CV, _F), lambda i: (i, 0)),
            pl.BlockSpec((_G, _F), lambda i: (0, 0)),
        ],
        out_shape=[
            jax.ShapeDtypeStruct((_N, _F), jnp.float32),
            jax.ShapeDtypeStruct((_G, _F), jnp.float32),
        ],
    )(g, h, w, bb, aa, batch3)


def _ffn_body(p_ref, w1_ref, b1_ref, w2_ref, b2_ref, o_ref):
    z = jnp.dot(p_ref[...], w1_ref[...], preferred_element_type=jnp.float32)
    z = jnp.maximum(z + b1_ref[...], 0.0)
    o_ref[...] = jnp.sum(z * w2_ref[...], axis=1, keepdims=True) + b2_ref[...]


def _ffn_call(pc, w1, b1, w2t, b2):
    d = _F * 5
    return pl.pallas_call(
        _ffn_body,
        in_specs=[
            pl.BlockSpec((_G, d), lambda: (0, 0)),
            pl.BlockSpec((d, d), lambda: (0, 0)),
            pl.BlockSpec((1, d), lambda: (0, 0)),
            pl.BlockSpec((1, d), lambda: (0, 0)),
            pl.BlockSpec((1, 1), lambda: (0, 0)),
        ],
        out_specs=pl.BlockSpec((_G, 1), lambda: (0, 0)),
        out_shape=jax.ShapeDtypeStruct((_G, 1), jnp.float32),
    )(pc, w1, b1, w2t, b2)


def kernel(x, batch, pre_W1, pre_b1, pre_a1, pre_W2, pre_b2, pre_a2,
           bn_gamma, bn_beta, act_a,
           conv0_W, conv0_b, conv1_W, conv1_b, conv2_W, conv2_b,
           conv3_W, conv3_b, ffn_W1, ffn_b1, ffn_W2, ffn_b2):
    batch = batch.astype(jnp.int32)
    r1 = lambda v: v.reshape(1, _F)

    xp = jnp.pad(x, ((0, _NP - _N), (0, 0)))
    xt = (-2.0 * xp).T
    brow = jnp.pad(batch, (0, _NP - _N), constant_values=-2).reshape(_NP, 1)
    bcol = jnp.pad(batch, (0, _NP - _N), constant_values=-1).reshape(1, _NP)
    batch3 = batch.reshape(_N // _BR_CV, 1, _BR_CV)

    sq = jnp.pad(jnp.sum(x * x, axis=1), (0, _NP - _N))

    bpad = jnp.pad(batch, (0, _NP - _N), constant_values=-1)
    blo = jnp.where(bpad >= 0, bpad, _BIG)
    bhi = jnp.where(bpad >= 0, bpad, -_BIG)
    rng = jnp.concatenate([
        jnp.min(blo.reshape(_NP // _BRK, _BRK), axis=1),
        jnp.max(bhi.reshape(_NP // _BRK, _BRK), axis=1),
        jnp.min(blo.reshape(_NCJ, _BCK), axis=1),
        jnp.max(bhi.reshape(_NCJ, _BCK), axis=1),
    ])

    nbr8 = _knn_call(xp, xt, sq.reshape(_NP, 1), sq.reshape(1, _NP),
                     brow, bcol, rng)
    nbr3 = nbr8[:, :_K].reshape(_NW, _NCH, _GROWS)

    h_pre, stats = _pre_call(x, pre_W1, r1(pre_b1), r1(pre_a1),
                             pre_W2, r1(pre_b2), r1(pre_a2))
    h, pool0 = _bn_call(h_pre, stats, r1(bn_gamma), r1(bn_beta), batch3)

    pools = [pool0]
    aa = r1(act_a)
    for w, bb in ((conv0_W, conv0_b), (conv1_W, conv1_b),
                  (conv2_W, conv2_b), (conv3_W, conv3_b)):
        g = _gather_sum_sc(h, nbr3)[:_N]
        h, pool = _conv_call(g, h, w, r1(bb), aa, batch3)
        pools.append(pool)

    pc = jnp.concatenate(pools, axis=1)
    return _ffn_call(pc, ffn_W1, ffn_b1.reshape(1, 5 * _F),
                     ffn_W2.reshape(1, 5 * _F), ffn_b2.reshape(1, 1))


# speedup vs baseline: 1.4544x; 1.3242x over previous
import functools

import jax
import jax.numpy as jnp
from jax import lax
from jax.experimental import pallas as pl
from jax.experimental.pallas import tpu as pltpu
from jax.experimental.pallas import tpu_sc as plsc

_N = 10000
_F = 128
_K = 6
_G = 16
_NP = 10240
_INF = float("inf")
_BIG = 2 ** 30


_BR_PRE = 2000


def _pre_body(x_ref, w1_ref, b1_ref, a1_ref, w2_ref, b2_ref, a2_ref,
              h_ref, st_ref):
    i = pl.program_id(0)
    xb = x_ref[...]
    h1 = jnp.dot(xb, w1_ref[...], preferred_element_type=jnp.float32) + b1_ref[...]
    h1 = jnp.where(h1 >= 0, h1, a1_ref[...] * h1)
    h2 = jnp.dot(h1, w2_ref[...], preferred_element_type=jnp.float32) + b2_ref[...]
    h2 = jnp.where(h2 >= 0, h2, a2_ref[...] * h2)
    h_ref[...] = h2
    s = jnp.sum(h2, axis=0, keepdims=True)
    ss = jnp.sum(h2 * h2, axis=0, keepdims=True)
    row = lax.broadcasted_iota(jnp.int32, (8, _F), 0)
    upd = jnp.where(row == 0, s, 0.0) + jnp.where(row == 1, ss, 0.0)

    @pl.when(i == 0)
    def _():
        st_ref[...] = upd

    @pl.when(i > 0)
    def _():
        st_ref[...] = st_ref[...] + upd


def _pre_call(x, w1, b1, a1, w2, b2, a2):
    nb = _N // _BR_PRE
    cspec = lambda shape: pl.BlockSpec(shape, lambda i: (0, 0))
    return pl.pallas_call(
        _pre_body,
        grid=(nb,),
        in_specs=[
            pl.BlockSpec((_BR_PRE, _F), lambda i: (i, 0)),
            cspec((_F, _F)), cspec((1, _F)), cspec((1, _F)),
            cspec((_F, _F)), cspec((1, _F)), cspec((1, _F)),
        ],
        out_specs=[
            pl.BlockSpec((_BR_PRE, _F), lambda i: (i, 0)),
            pl.BlockSpec((8, _F), lambda i: (0, 0)),
        ],
        out_shape=[
            jax.ShapeDtypeStruct((_N, _F), jnp.float32),
            jax.ShapeDtypeStruct((8, _F), jnp.float32),
        ],
    )(x, w1, b1, a1, w2, b2, a2)


_BR_BN = 2000


def _bn_body(hp_ref, st_ref, gm_ref, bt_ref, bidx_ref, h0_ref, pool_ref):
    i = pl.program_id(0)
    st = st_ref[...]
    mu = st[0:1, :] * (1.0 / _N)
    ex2 = st[1:2, :] * (1.0 / _N)
    var = ex2 - mu * mu
    scale = lax.rsqrt(var + 1e-5) * gm_ref[...]
    h0 = (hp_ref[...] - mu) * scale + bt_ref[...]
    h0_ref[...] = h0
    b = bidx_ref[0]
    oh = (b == lax.broadcasted_iota(jnp.int32, (_G, _BR_BN), 0)).astype(jnp.float32)
    pool = lax.dot_general(oh, h0, (((1,), (0,)), ((), ())),
                           preferred_element_type=jnp.float32)

    @pl.when(i == 0)
    def _():
        pool_ref[...] = pool

    @pl.when(i > 0)
    def _():
        pool_ref[...] = pool_ref[...] + pool


def _bn_call(h_pre, stats, gamma, beta, batch3):
    nb = _N // _BR_BN
    return pl.pallas_call(
        _bn_body,
        grid=(nb,),
        in_specs=[
            pl.BlockSpec((_BR_BN, _F), lambda i: (i, 0)),
            pl.BlockSpec((8, _F), lambda i: (0, 0)),
            pl.BlockSpec((1, _F), lambda i: (0, 0)),
            pl.BlockSpec((1, _F), lambda i: (0, 0)),
            pl.BlockSpec((1, 1, _BR_BN), lambda i: (i, 0, 0)),
        ],
        out_specs=[
            pl.BlockSpec((_BR_BN, _F), lambda i: (i, 0)),
            pl.BlockSpec((_G, _F), lambda i: (0, 0)),
        ],
        out_shape=[
            jax.ShapeDtypeStruct((_N, _F), jnp.float32),
            jax.ShapeDtypeStruct((_G, _F), jnp.float32),
        ],
    )(h_pre, stats, gamma, beta, batch3)


_BRK = 256
_BCK = 2048
_NCJ = _NP // _BCK


_FBIG = 1e9


def _knn_body(rng_ref, xr_ref, xt_ref, sqr_ref, sqc_ref, brow_ref, bcol_ref,
              nbr_ref, rv_ref, ri_ref):
    i = pl.program_id(0)
    j = pl.program_id(1)

    @pl.when(j == 0)
    def _():
        rv_ref[...] = jnp.full((_BRK, 8), _INF, jnp.float32)
        ri_ref[...] = jnp.full((_BRK, 8), _FBIG, jnp.float32)

    nbi = _NP // _BRK
    rel = ((rng_ref[2 * nbi + j] <= rng_ref[nbi + i])
           & (rng_ref[2 * nbi + _NCJ + j] >= rng_ref[i]))

    @pl.when(rel)
    def _():
        xr = xr_ref[...]
        xt = xt_ref[...]
        sqr = sqr_ref[...]
        sqc = sqc_ref[...]
        d = lax.dot_general(xr, xt, (((1,), (0,)), ((), ())),
                            preferred_element_type=jnp.float32)
        d2 = (sqr + sqc) + d
        coli = lax.broadcasted_iota(jnp.int32, (_BRK, _BCK), 1)
        rowi = lax.broadcasted_iota(jnp.int32, (_BRK, _BCK), 0)
        colg = (coli + j * _BCK).astype(jnp.float32)
        mask = (brow_ref[...] != bcol_ref[...]) | ((coli - rowi) == (i * _BRK - j * _BCK))
        d2 = jnp.where(mask, _INF, d2)

        cv, ci = [], []
        for _ in range(_K):
            m = jnp.min(d2, axis=1, keepdims=True)
            sel = jnp.where(d2 == m, colg, _FBIG)
            am = jnp.min(sel, axis=1, keepdims=True)
            cv.append(m)
            ci.append(am)
            d2 = jnp.where(sel == am, _INF, d2)

        allv = jnp.concatenate(cv + [rv_ref[...]], axis=1)
        alli = jnp.concatenate(ci + [ri_ref[...]], axis=1)
        nv, ni = [], []
        for _ in range(_K):
            m = jnp.min(allv, axis=1, keepdims=True)
            sel = jnp.where(allv == m, alli, _FBIG)
            am = jnp.min(sel, axis=1, keepdims=True)
            nv.append(m)
            ni.append(am)
            allv = jnp.where(sel == am, _INF, allv)

        rv_ref[:, 0:_K] = jnp.concatenate(nv, axis=1)
        ri_ref[:, 0:_K] = jnp.concatenate(ni, axis=1)

    @pl.when(j == _NCJ - 1)
    def _():
        ri = ri_ref[...]
        idx8 = jnp.concatenate(
            [ri[:, 0:_K], ri[:, _K - 1:_K], ri[:, _K - 1:_K]], axis=1)
        nbr_ref[...] = jnp.minimum(idx8, _N - 1).astype(jnp.int32)


def _knn_call(xp, xt, sqr, sqc, brow, bcol, rng):
    nbi = _NP // _BRK
    return pl.pallas_call(
        _knn_body,
        grid_spec=pltpu.PrefetchScalarGridSpec(
            num_scalar_prefetch=1,
            grid=(nbi, _NCJ),
            in_specs=[
                pl.BlockSpec((_BRK, _F), lambda i, j, rng: (i, 0)),
                pl.BlockSpec((_F, _BCK), lambda i, j, rng: (0, j)),
                pl.BlockSpec((_BRK, 1), lambda i, j, rng: (i, 0)),
                pl.BlockSpec((1, _BCK), lambda i, j, rng: (0, j)),
                pl.BlockSpec((_BRK, 1), lambda i, j, rng: (i, 0)),
                pl.BlockSpec((1, _BCK), lambda i, j, rng: (0, j)),
            ],
            out_specs=pl.BlockSpec((_BRK, 8), lambda i, j, rng: (i, 0)),
            scratch_shapes=[
                pltpu.VMEM((_BRK, 8), jnp.float32),
                pltpu.VMEM((_BRK, 8), jnp.float32),
            ],
        ),
        out_shape=jax.ShapeDtypeStruct((_NP, 8), jnp.int32),
    )(rng, xp, xt, sqr, sqc, brow, bcol)


_NW = 32
_BW = _NP // _NW
_GN = 16
_GROWS = _GN * _K
_NCH = _BW // _GN


def _gather_sum_sc(h, nbr3):
    mesh = plsc.VectorSubcoreMesh(core_axis_name="c", subcore_axis_name="s")

    @functools.partial(
        pl.kernel,
        out_type=jax.ShapeDtypeStruct((_NP, _F), jnp.float32),
        mesh=mesh,
        scratch_types=[
            pltpu.VMEM((_NCH, _GROWS), jnp.int32),
            pltpu.VMEM((4, _GROWS, _F), jnp.float32),
            pltpu.VMEM((_BW, _F), jnp.float32),
            pltpu.SemaphoreType.DMA,
            pltpu.SemaphoreType.DMA,
            pltpu.SemaphoreType.DMA,
            pltpu.SemaphoreType.DMA,
        ],
    )
    def gk(h_hbm, nbr_hbm, out_hbm, idx_v, rows_v, acc_v, sem0, sem1, sem2, sem3):
        wid = lax.axis_index("s") * 2 + lax.axis_index("c")
        base = wid * _BW
        pltpu.sync_copy(nbr_hbm.at[wid], idx_v)
        sems = [sem0, sem1, sem2, sem3]
        ring = 4
        handles = [None] * ring
        for p in range(ring):
            handles[p] = pltpu.async_copy(
                h_hbm.at[idx_v.at[p]], rows_v.at[p], sems[p])
        for c in range(_NCH):
            b = c % ring
            handles[b].wait()

            def body(nn, _, _b=b, _c=c):
                r0 = nn * _K
                for f8 in range(_F // 16):
                    sl = pl.ds(f8 * 16, 16)
                    v = rows_v[_b, r0, sl]
                    for k in range(1, _K):
                        v = v + rows_v[_b, r0 + k, sl]
                    acc_v[_c * _GN + nn, sl] = v
                return 0

            lax.fori_loop(0, _GN, body, 0)
            if c + ring < _NCH:
                handles[b] = pltpu.async_copy(
                    h_hbm.at[idx_v.at[c + ring]], rows_v.at[b], sems[b])
        pltpu.sync_copy(acc_v, out_hbm.at[pl.ds(base, _BW)])

    return gk(h, nbr3)


_BR_CV = 2000


def _conv_body(g_ref, h_ref, w_ref, b_ref, a_ref, bidx_ref, hn_ref, pool_ref):
    i = pl.program_id(0)
    agg = jnp.dot(g_ref[...], w_ref[...], preferred_element_type=jnp.float32)
    hn = agg + (_K * 1.0) * b_ref[...] + h_ref[...]
    a = a_ref[...]
    hn = jnp.where(hn >= 0, hn, a * hn)
    hn_ref[...] = hn
    b = bidx_ref[0]
    oh = (b == lax.broadcasted_iota(jnp.int32, (_G, _BR_CV), 0)).astype(jnp.float32)
    pool = lax.dot_general(oh, hn, (((1,), (0,)), ((), ())),
                           preferred_element_type=jnp.float32)

    @pl.when(i == 0)
    def _():
        pool_ref[...] = pool

    @pl.when(i > 0)
    def _():
        pool_ref[...] = pool_ref[...] + pool


def _conv_call(g, h, w, bb, aa, batch3):
    nb = _N // _BR_CV
    return pl.pallas_call(
        _conv_body,
        grid=(nb,),
        in_specs=[
            pl.BlockSpec((_BR_CV, _F), lambda i: (i, 0)),
            pl.BlockSpec((_BR_CV, _F), lambda i: (i, 0)),
            pl.BlockSpec((_F, _F), lambda i: (0, 0)),
            pl.BlockSpec((1, _F), lambda i: (0, 0)),
            pl.BlockSpec((1, _F), lambda i: (0, 0)),
            pl.BlockSpec((1, 1, _BR_CV), lambda i: (i, 0, 0)),
        ],
        out_specs=[
            pl.BlockSpec((_BR_CV, _F), lambda i: (i, 0)),
            pl.BlockSpec((_G, _F), lambda i: (0, 0)),
        ],
        out_shape=[
            jax.ShapeDtypeStruct((_N, _F), jnp.float32),
            jax.ShapeDtypeStruct((_G, _F), jnp.float32),
        ],
    )(g, h, w, bb, aa, batch3)


def _ffn_body(p_ref, w1_ref, b1_ref, w2_ref, b2_ref, o_ref):
    z = jnp.dot(p_ref[...], w1_ref[...], preferred_element_type=jnp.float32)
    z = jnp.maximum(z + b1_ref[...], 0.0)
    o_ref[...] = jnp.sum(z * w2_ref[...], axis=1, keepdims=True) + b2_ref[...]


def _ffn_call(pc, w1, b1, w2t, b2):
    d = _F * 5
    return pl.pallas_call(
        _ffn_body,
        in_specs=[
            pl.BlockSpec((_G, d), lambda: (0, 0)),
            pl.BlockSpec((d, d), lambda: (0, 0)),
            pl.BlockSpec((1, d), lambda: (0, 0)),
            pl.BlockSpec((1, d), lambda: (0, 0)),
            pl.BlockSpec((1, 1), lambda: (0, 0)),
        ],
        out_specs=pl.BlockSpec((_G, 1), lambda: (0, 0)),
        out_shape=jax.ShapeDtypeStruct((_G, 1), jnp.float32),
    )(pc, w1, b1, w2t, b2)


def kernel(x, batch, pre_W1, pre_b1, pre_a1, pre_W2, pre_b2, pre_a2,
           bn_gamma, bn_beta, act_a,
           conv0_W, conv0_b, conv1_W, conv1_b, conv2_W, conv2_b,
           conv3_W, conv3_b, ffn_W1, ffn_b1, ffn_W2, ffn_b2):
    batch = batch.astype(jnp.int32)
    r1 = lambda v: v.reshape(1, _F)

    xp = jnp.pad(x, ((0, _NP - _N), (0, 0)))
    xt = (-2.0 * xp).T
    brow = jnp.pad(batch, (0, _NP - _N), constant_values=-2).reshape(_NP, 1)
    bcol = jnp.pad(batch, (0, _NP - _N), constant_values=-1).reshape(1, _NP)
    batch3 = batch.reshape(_N // _BR_CV, 1, _BR_CV)

    sq = jnp.pad(jnp.sum(x * x, axis=1), (0, _NP - _N))

    bpad = jnp.pad(batch, (0, _NP - _N), constant_values=-1)
    blo = jnp.where(bpad >= 0, bpad, _BIG)
    bhi = jnp.where(bpad >= 0, bpad, -_BIG)
    rng = jnp.concatenate([
        jnp.min(blo.reshape(_NP // _BRK, _BRK), axis=1),
        jnp.max(bhi.reshape(_NP // _BRK, _BRK), axis=1),
        jnp.min(blo.reshape(_NCJ, _BCK), axis=1),
        jnp.max(bhi.reshape(_NCJ, _BCK), axis=1),
    ])

    nbr8 = _knn_call(xp, xt, sq.reshape(_NP, 1), sq.reshape(1, _NP),
                     brow, bcol, rng)
    rows = lax.broadcasted_iota(jnp.int32, (_NP, _K), 0)
    cols = lax.broadcasted_iota(jnp.int32, (_NP, _K), 1)
    nbr = jnp.where(rows < _N, nbr8[:, :_K], (rows * _K + cols) % _N)
    nbr3 = nbr.reshape(_NW, _NCH, _GROWS)

    h_pre, stats = _pre_call(x, pre_W1, r1(pre_b1), r1(pre_a1),
                             pre_W2, r1(pre_b2), r1(pre_a2))
    h, pool0 = _bn_call(h_pre, stats, r1(bn_gamma), r1(bn_beta), batch3)

    pools = [pool0]
    aa = r1(act_a)
    for w, bb in ((conv0_W, conv0_b), (conv1_W, conv1_b),
                  (conv2_W, conv2_b), (conv3_W, conv3_b)):
        g = _gather_sum_sc(h, nbr3)[:_N]
        h, pool = _conv_call(g, h, w, r1(bb), aa, batch3)
        pools.append(pool)

    pc = jnp.concatenate(pools, axis=1)
    return _ffn_call(pc, ffn_W1, ffn_b1.reshape(1, 5 * _F),
                     ffn_W2.reshape(1, 5 * _F), ffn_b2.reshape(1, 1))


# speedup vs baseline: 1.4555x; 1.0008x over previous
import functools

import jax
import jax.numpy as jnp
from jax import lax
from jax.experimental import pallas as pl
from jax.experimental.pallas import tpu as pltpu
from jax.experimental.pallas import tpu_sc as plsc

_N = 10000
_F = 128
_K = 6
_G = 16
_NP = 10240
_INF = float("inf")
_BIG = 2 ** 30


_BR_PRE = 2000


def _pre_body(x_ref, w1_ref, b1_ref, a1_ref, w2_ref, b2_ref, a2_ref,
              h_ref, st_ref):
    i = pl.program_id(0)
    xb = x_ref[...]
    h1 = jnp.dot(xb, w1_ref[...], preferred_element_type=jnp.float32) + b1_ref[...]
    h1 = jnp.where(h1 >= 0, h1, a1_ref[...] * h1)
    h2 = jnp.dot(h1, w2_ref[...], preferred_element_type=jnp.float32) + b2_ref[...]
    h2 = jnp.where(h2 >= 0, h2, a2_ref[...] * h2)
    h_ref[...] = h2
    s = jnp.sum(h2, axis=0, keepdims=True)
    ss = jnp.sum(h2 * h2, axis=0, keepdims=True)
    row = lax.broadcasted_iota(jnp.int32, (8, _F), 0)
    upd = jnp.where(row == 0, s, 0.0) + jnp.where(row == 1, ss, 0.0)

    @pl.when(i == 0)
    def _():
        st_ref[...] = upd

    @pl.when(i > 0)
    def _():
        st_ref[...] = st_ref[...] + upd


def _pre_call(x, w1, b1, a1, w2, b2, a2):
    nb = _N // _BR_PRE
    cspec = lambda shape: pl.BlockSpec(shape, lambda i: (0, 0))
    return pl.pallas_call(
        _pre_body,
        grid=(nb,),
        in_specs=[
            pl.BlockSpec((_BR_PRE, _F), lambda i: (i, 0)),
            cspec((_F, _F)), cspec((1, _F)), cspec((1, _F)),
            cspec((_F, _F)), cspec((1, _F)), cspec((1, _F)),
        ],
        out_specs=[
            pl.BlockSpec((_BR_PRE, _F), lambda i: (i, 0)),
            pl.BlockSpec((8, _F), lambda i: (0, 0)),
        ],
        out_shape=[
            jax.ShapeDtypeStruct((_N, _F), jnp.float32),
            jax.ShapeDtypeStruct((8, _F), jnp.float32),
        ],
    )(x, w1, b1, a1, w2, b2, a2)


_BR_BN = 2000


def _bn_body(hp_ref, st_ref, gm_ref, bt_ref, bidx_ref, h0_ref, pool_ref):
    i = pl.program_id(0)
    st = st_ref[...]
    mu = st[0:1, :] * (1.0 / _N)
    ex2 = st[1:2, :] * (1.0 / _N)
    var = ex2 - mu * mu
    scale = lax.rsqrt(var + 1e-5) * gm_ref[...]
    h0 = (hp_ref[...] - mu) * scale + bt_ref[...]
    h0_ref[...] = h0
    b = bidx_ref[0]
    oh = (b == lax.broadcasted_iota(jnp.int32, (_G, _BR_BN), 0)).astype(jnp.float32)
    pool = lax.dot_general(oh, h0, (((1,), (0,)), ((), ())),
                           preferred_element_type=jnp.float32)

    @pl.when(i == 0)
    def _():
        pool_ref[...] = pool

    @pl.when(i > 0)
    def _():
        pool_ref[...] = pool_ref[...] + pool


def _bn_call(h_pre, stats, gamma, beta, batch3):
    nb = _N // _BR_BN
    return pl.pallas_call(
        _bn_body,
        grid=(nb,),
        in_specs=[
            pl.BlockSpec((_BR_BN, _F), lambda i: (i, 0)),
            pl.BlockSpec((8, _F), lambda i: (0, 0)),
            pl.BlockSpec((1, _F), lambda i: (0, 0)),
            pl.BlockSpec((1, _F), lambda i: (0, 0)),
            pl.BlockSpec((1, 1, _BR_BN), lambda i: (i, 0, 0)),
        ],
        out_specs=[
            pl.BlockSpec((_BR_BN, _F), lambda i: (i, 0)),
            pl.BlockSpec((_G, _F), lambda i: (0, 0)),
        ],
        out_shape=[
            jax.ShapeDtypeStruct((_N, _F), jnp.float32),
            jax.ShapeDtypeStruct((_G, _F), jnp.float32),
        ],
    )(h_pre, stats, gamma, beta, batch3)


_BRK = 256
_BCK = 2048
_NCJ = _NP // _BCK


_FBIG = 1e9


def _knn_body(rng_ref, xr_ref, xt_ref, sqr_ref, sqc_ref, brow_ref, bcol_ref,
              nbr_ref, rv_ref, ri_ref):
    i = pl.program_id(0)
    j = pl.program_id(1)

    @pl.when(j == 0)
    def _():
        rv_ref[...] = jnp.full((_BRK, 8), _INF, jnp.float32)
        ri_ref[...] = jnp.full((_BRK, 8), _FBIG, jnp.float32)

    nbi = _NP // _BRK
    rel = ((rng_ref[2 * nbi + j] <= rng_ref[nbi + i])
           & (rng_ref[2 * nbi + _NCJ + j] >= rng_ref[i]))

    @pl.when(rel)
    def _():
        xr = xr_ref[...]
        xt = xt_ref[...]
        sqr = sqr_ref[...]
        sqc = sqc_ref[...]
        d = lax.dot_general(xr, xt, (((1,), (0,)), ((), ())),
                            preferred_element_type=jnp.float32)
        d2 = (sqr + sqc) + d
        coli = lax.broadcasted_iota(jnp.int32, (_BRK, _BCK), 1)
        rowi = lax.broadcasted_iota(jnp.int32, (_BRK, _BCK), 0)
        colg = (coli + j * _BCK).astype(jnp.float32)
        mask = (brow_ref[...] != bcol_ref[...]) | ((coli - rowi) == (i * _BRK - j * _BCK))
        d2 = jnp.where(mask, _INF, d2)

        cv, ci = [], []
        for _ in range(_K):
            m = jnp.min(d2, axis=1, keepdims=True)
            sel = jnp.where(d2 == m, colg, _FBIG)
            am = jnp.min(sel, axis=1, keepdims=True)
            cv.append(m)
            ci.append(am)
            d2 = jnp.where(sel == am, _INF, d2)

        allv = jnp.concatenate(cv + [rv_ref[...]], axis=1)
        alli = jnp.concatenate(ci + [ri_ref[...]], axis=1)
        nv, ni = [], []
        for _ in range(_K):
            m = jnp.min(allv, axis=1, keepdims=True)
            sel = jnp.where(allv == m, alli, _FBIG)
            am = jnp.min(sel, axis=1, keepdims=True)
            nv.append(m)
            ni.append(am)
            allv = jnp.where(sel == am, _INF, allv)

        rv_ref[:, 0:_K] = jnp.concatenate(nv, axis=1)
        ri_ref[:, 0:_K] = jnp.concatenate(ni, axis=1)

    @pl.when(j == _NCJ - 1)
    def _():
        ri = ri_ref[...]
        idx8 = jnp.concatenate(
            [ri[:, 0:_K], ri[:, _K - 1:_K], ri[:, _K - 1:_K]], axis=1)
        nbr_ref[...] = jnp.minimum(idx8, _N - 1).astype(jnp.int32)


def _knn_call(xp, xt, sqr, sqc, brow, bcol, rng):
    nbi = _NP // _BRK
    return pl.pallas_call(
        _knn_body,
        grid_spec=pltpu.PrefetchScalarGridSpec(
            num_scalar_prefetch=1,
            grid=(nbi, _NCJ),
            in_specs=[
                pl.BlockSpec((_BRK, _F), lambda i, j, rng: (i, 0)),
                pl.BlockSpec((_F, _BCK), lambda i, j, rng: (0, j)),
                pl.BlockSpec((_BRK, 1), lambda i, j, rng: (i, 0)),
                pl.BlockSpec((1, _BCK), lambda i, j, rng: (0, j)),
                pl.BlockSpec((_BRK, 1), lambda i, j, rng: (i, 0)),
                pl.BlockSpec((1, _BCK), lambda i, j, rng: (0, j)),
            ],
            out_specs=pl.BlockSpec((_BRK, 8), lambda i, j, rng: (i, 0)),
            scratch_shapes=[
                pltpu.VMEM((_BRK, 8), jnp.float32),
                pltpu.VMEM((_BRK, 8), jnp.float32),
            ],
        ),
        out_shape=jax.ShapeDtypeStruct((_NP, 8), jnp.int32),
    )(rng, xp, xt, sqr, sqc, brow, bcol)


_NW = 32
_BW = _NP // _NW
_GN = 16
_GROWS = _GN * _K
_NCH = _BW // _GN


def _gather_sum_sc(h, nbr3):
    mesh = plsc.VectorSubcoreMesh(core_axis_name="c", subcore_axis_name="s")

    @functools.partial(
        pl.kernel,
        out_type=jax.ShapeDtypeStruct((_NP, _F), jnp.float32),
        mesh=mesh,
        scratch_types=[
            pltpu.VMEM((_NCH, _GROWS), jnp.int32),
            pltpu.VMEM((4, _GROWS, _F), jnp.float32),
            pltpu.VMEM((_BW, _F), jnp.float32),
            pltpu.SemaphoreType.DMA,
            pltpu.SemaphoreType.DMA,
            pltpu.SemaphoreType.DMA,
            pltpu.SemaphoreType.DMA,
        ],
    )
    def gk(h_hbm, nbr_hbm, out_hbm, idx_v, rows_v, acc_v, sem0, sem1, sem2, sem3):
        wid = lax.axis_index("s") * 2 + lax.axis_index("c")
        base = wid * _BW
        pltpu.sync_copy(nbr_hbm.at[wid], idx_v)
        sems = [sem0, sem1, sem2, sem3]
        ring = 4
        handles = [None] * ring
        for p in range(ring):
            handles[p] = pltpu.async_copy(
                h_hbm.at[idx_v.at[p]], rows_v.at[p], sems[p])
        for c in range(_NCH):
            b = c % ring
            handles[b].wait()

            def body(nn, _, _b=b, _c=c):
                r0 = nn * _K
                for f8 in range(_F // 16):
                    sl = pl.ds(f8 * 16, 16)
                    v = rows_v[_b, r0, sl]
                    for k in range(1, _K):
                        v = v + rows_v[_b, r0 + k, sl]
                    acc_v[_c * _GN + nn, sl] = v
                return 0

            lax.fori_loop(0, _GN, body, 0)
            if c + ring < _NCH:
                handles[b] = pltpu.async_copy(
                    h_hbm.at[idx_v.at[c + ring]], rows_v.at[b], sems[b])
        pltpu.sync_copy(acc_v, out_hbm.at[pl.ds(base, _BW)])

    return gk(h, nbr3)


_BR_CV = 2000


def _conv_body(g_ref, h_ref, w_ref, b_ref, a_ref, bidx_ref, hn_ref, pool_ref):
    i = pl.program_id(0)
    agg = jnp.dot(g_ref[...], w_ref[...], preferred_element_type=jnp.float32)
    hn = agg + (_K * 1.0) * b_ref[...] + h_ref[...]
    a = a_ref[...]
    hn = jnp.where(hn >= 0, hn, a * hn)
    hn_ref[...] = hn
    b = bidx_ref[0]
    oh = (b == lax.broadcasted_iota(jnp.int32, (_G, _BR_CV), 0)).astype(jnp.float32)
    pool = lax.dot_general(oh, hn, (((1,), (0,)), ((), ())),
                           preferred_element_type=jnp.float32)

    @pl.when(i == 0)
    def _():
        pool_ref[...] = pool

    @pl.when(i > 0)
    def _():
        pool_ref[...] = pool_ref[...] + pool


def _conv_call(g, h, w, bb, aa, batch3):
    nb = _N // _BR_CV
    return pl.pallas_call(
        _conv_body,
        grid=(nb,),
        in_specs=[
            pl.BlockSpec((_BR_CV, _F), lambda i: (i, 0)),
            pl.BlockSpec((_BR_CV, _F), lambda i: (i, 0)),
            pl.BlockSpec((_F, _F), lambda i: (0, 0)),
            pl.BlockSpec((1, _F), lambda i: (0, 0)),
            pl.BlockSpec((1, _F), lambda i: (0, 0)),
            pl.BlockSpec((1, 1, _BR_CV), lambda i: (i, 0, 0)),
        ],
        out_specs=[
            pl.BlockSpec((_BR_CV, _F), lambda i: (i, 0)),
            pl.BlockSpec((_G, _F), lambda i: (0, 0)),
        ],
        out_shape=[
            jax.ShapeDtypeStruct((_N, _F), jnp.float32),
            jax.ShapeDtypeStruct((_G, _F), jnp.float32),
        ],
    )(g, h, w, bb, aa, batch3)


def _ffn_body(p_ref, w1_ref, b1_ref, w2_ref, b2_ref, o_ref):
    z = jnp.dot(p_ref[...], w1_ref[...], preferred_element_type=jnp.float32)
    z = jnp.maximum(z + b1_ref[...], 0.0)
    o_ref[...] = jnp.sum(z * w2_ref[...], axis=1, keepdims=True) + b2_ref[...]


def _ffn_call(pc, w1, b1, w2t, b2):
    d = _F * 5
    return pl.pallas_call(
        _ffn_body,
        in_specs=[
            pl.BlockSpec((_G, d), lambda: (0, 0)),
            pl.BlockSpec((d, d), lambda: (0, 0)),
            pl.BlockSpec((1, d), lambda: (0, 0)),
            pl.BlockSpec((1, d), lambda: (0, 0)),
            pl.BlockSpec((1, 1), lambda: (0, 0)),
        ],
        out_specs=pl.BlockSpec((_G, 1), lambda: (0, 0)),
        out_shape=jax.ShapeDtypeStruct((_G, 1), jnp.float32),
    )(pc, w1, b1, w2t, b2)


def kernel(x, batch, pre_W1, pre_b1, pre_a1, pre_W2, pre_b2, pre_a2,
           bn_gamma, bn_beta, act_a,
           conv0_W, conv0_b, conv1_W, conv1_b, conv2_W, conv2_b,
           conv3_W, conv3_b, ffn_W1, ffn_b1, ffn_W2, ffn_b2):
    batch = batch.astype(jnp.int32)
    r1 = lambda v: v.reshape(1, _F)

    xp = jnp.pad(x, ((0, _NP - _N), (0, 0)))
    xt = (-2.0 * xp).T
    brow = jnp.pad(batch, (0, _NP - _N), constant_values=-2).reshape(_NP, 1)
    bcol = jnp.pad(batch, (0, _NP - _N), constant_values=-1).reshape(1, _NP)
    batch3 = batch.reshape(_N // _BR_CV, 1, _BR_CV)

    sq = jnp.pad(lax.optimization_barrier(jnp.sum(x * x, axis=1)),
                 (0, _NP - _N))

    bpad = jnp.pad(batch, (0, _NP - _N), constant_values=-1)
    blo = jnp.where(bpad >= 0, bpad, _BIG)
    bhi = jnp.where(bpad >= 0, bpad, -_BIG)
    rng = jnp.concatenate([
        jnp.min(blo.reshape(_NP // _BRK, _BRK), axis=1),
        jnp.max(bhi.reshape(_NP // _BRK, _BRK), axis=1),
        jnp.min(blo.reshape(_NCJ, _BCK), axis=1),
        jnp.max(bhi.reshape(_NCJ, _BCK), axis=1),
    ])

    nbr8 = _knn_call(xp, xt, sq.reshape(_NP, 1), sq.reshape(1, _NP),
                     brow, bcol, rng)
    rows = lax.broadcasted_iota(jnp.int32, (_NP, _K), 0)
    cols = lax.broadcasted_iota(jnp.int32, (_NP, _K), 1)
    nbr = jnp.where(rows < _N, nbr8[:, :_K], (rows * _K + cols) % _N)
    nbr3 = nbr.reshape(_NW, _NCH, _GROWS)

    h_pre, stats = _pre_call(x, pre_W1, r1(pre_b1), r1(pre_a1),
                             pre_W2, r1(pre_b2), r1(pre_a2))
    h, pool0 = _bn_call(h_pre, stats, r1(bn_gamma), r1(bn_beta), batch3)

    pools = [pool0]
    aa = r1(act_a)
    for w, bb in ((conv0_W, conv0_b), (conv1_W, conv1_b),
                  (conv2_W, conv2_b), (conv3_W, conv3_b)):
        g = _gather_sum_sc(h, nbr3)[:_N]
        h, pool = _conv_call(g, h, w, r1(bb), aa, batch3)
        pools.append(pool)

    pc = jnp.concatenate(pools, axis=1)
    return _ffn_call(pc, ffn_W1, ffn_b1.reshape(1, 5 * _F),
                     ffn_W2.reshape(1, 5 * _F), ffn_b2.reshape(1, 1))
